# Initial kernel scaffold; baseline (speedup 1.0000x reference)
import jax
import jax.numpy as jnp
from jax.experimental import pallas as pl


def kernel(hidden_states, W_gate):
    raise NotImplementedError("write your pallas kernel here")


# speedup vs baseline: 2.0856x; 2.0856x over previous
import functools

import jax
import jax.numpy as jnp
from jax import lax
from jax.experimental import pallas as pl
from jax.experimental.pallas import tpu as pltpu

_NUM_EXPERTS = 64
_TOP_K = 2
_ALPHA = 0.01
_TILE = 1024


def _router_body(x_ref, wt_ref, w_out, e_out, loss_out, psum, cnt, *, n_steps,
                 num_tokens):
    pid = pl.program_id(0)

    @pl.when(pid == 0)
    def _init():
        psum[...] = jnp.zeros_like(psum)
        cnt[...] = jnp.zeros_like(cnt)

    logits = jnp.dot(x_ref[...], wt_ref[...],
                     preferred_element_type=jnp.float32)
    col = lax.broadcasted_iota(jnp.int32, logits.shape, 1)

    m1 = jnp.max(logits, axis=-1, keepdims=True)
    i1 = jnp.min(jnp.where(logits == m1, col, _NUM_EXPERTS), axis=-1,
                 keepdims=True)
    masked = jnp.where(col == i1, -jnp.inf, logits)
    m2 = jnp.max(masked, axis=-1, keepdims=True)
    i2 = jnp.min(jnp.where(masked == m2, col, _NUM_EXPERTS), axis=-1,
                 keepdims=True)

    t = jnp.exp(m2 - m1)
    w1 = 1.0 / (1.0 + t)
    w_out[...] = jnp.concatenate([w1, 1.0 - w1], axis=1)
    e_out[...] = jnp.concatenate([i1, i2], axis=1)

    e = jnp.exp(logits - m1)
    z = jnp.sum(e, axis=-1, keepdims=True)
    psum[...] += jnp.sum(e / z, axis=0, keepdims=True)
    hits = (col == i1).astype(jnp.float32) + (col == i2).astype(jnp.float32)
    cnt[...] += jnp.sum(hits, axis=0, keepdims=True)

    @pl.when(pid == n_steps - 1)
    def _fin():
        scale = _ALPHA * _NUM_EXPERTS / (num_tokens * num_tokens)
        loss_out[...] = scale * jnp.sum(psum[...] * cnt[...],
                                        keepdims=True)


def kernel(hidden_states, W_gate):
    batch, seq, hidden = hidden_states.shape
    num_tokens = batch * seq
    x = hidden_states.reshape(num_tokens, hidden)
    wt = W_gate.T
    n_steps = num_tokens // _TILE

    grid = (n_steps,)
    weights, experts, loss = pl.pallas_call(
        functools.partial(_router_body, n_steps=n_steps,
                          num_tokens=num_tokens),
        grid=grid,
        in_specs=[
            pl.BlockSpec((_TILE, hidden), lambda i: (i, 0)),
            pl.BlockSpec((hidden, _NUM_EXPERTS), lambda i: (0, 0)),
        ],
        out_specs=[
            pl.BlockSpec((_TILE, _TOP_K), lambda i: (i, 0)),
            pl.BlockSpec((_TILE, _TOP_K), lambda i: (i, 0)),
            pl.BlockSpec((1, 1), lambda i: (0, 0)),
        ],
        out_shape=[
            jax.ShapeDtypeStruct((num_tokens, _TOP_K), jnp.float32),
            jax.ShapeDtypeStruct((num_tokens, _TOP_K), jnp.int32),
            jax.ShapeDtypeStruct((1, 1), jnp.float32),
        ],
        scratch_shapes=[
            pltpu.VMEM((1, _NUM_EXPERTS), jnp.float32),
            pltpu.VMEM((1, _NUM_EXPERTS), jnp.float32),
        ],
    )(x, wt)

    return (weights.reshape(batch, seq, _TOP_K),
            experts.reshape(batch, seq, _TOP_K),
            loss[0, 0])


# speedup vs baseline: 2.3368x; 1.1205x over previous
import functools

import jax
import jax.numpy as jnp
from jax import lax
from jax.experimental import pallas as pl
from jax.experimental.pallas import tpu as pltpu

_NUM_EXPERTS = 64
_TOP_K = 2
_ALPHA = 0.01
_TILE = 2048


def _router_body(x_ref, wt_ref, w_out, e_out, loss_out, psum, cnt, *, n_steps,
                 num_tokens):
    pid = pl.program_id(0)

    @pl.when(pid == 0)
    def _init():
        psum[...] = jnp.zeros_like(psum)
        cnt[...] = jnp.zeros_like(cnt)

    logits = jnp.dot(x_ref[...], wt_ref[...],
                     preferred_element_type=jnp.float32)
    col = lax.broadcasted_iota(jnp.int32, logits.shape, 1)

    m1 = jnp.max(logits, axis=-1, keepdims=True)
    i1 = jnp.min(jnp.where(logits == m1, col, _NUM_EXPERTS), axis=-1,
                 keepdims=True)
    masked = jnp.where(col == i1, -jnp.inf, logits)
    m2 = jnp.max(masked, axis=-1, keepdims=True)
    i2 = jnp.min(jnp.where(masked == m2, col, _NUM_EXPERTS), axis=-1,
                 keepdims=True)

    t = jnp.exp(m2 - m1)
    w1 = 1.0 / (1.0 + t)
    w_out[...] = jnp.concatenate([w1, 1.0 - w1], axis=1)
    e_out[...] = jnp.concatenate([i1, i2], axis=1)

    e = jnp.exp(logits - m1)
    z = jnp.sum(e, axis=-1, keepdims=True)
    psum[...] += jnp.sum(e / z, axis=0, keepdims=True)
    hits = (col == i1).astype(jnp.float32) + (col == i2).astype(jnp.float32)
    cnt[...] += jnp.sum(hits, axis=0, keepdims=True)

    @pl.when(pid == n_steps - 1)
    def _fin():
        scale = _ALPHA * _NUM_EXPERTS / (num_tokens * num_tokens)
        loss_out[...] = scale * jnp.sum(psum[...] * cnt[...],
                                        keepdims=True)


def kernel(hidden_states, W_gate):
    batch, seq, hidden = hidden_states.shape
    num_tokens = batch * seq
    x = hidden_states.reshape(num_tokens, hidden)
    wt = W_gate.T
    n_steps = num_tokens // _TILE

    grid = (n_steps,)
    weights, experts, loss = pl.pallas_call(
        functools.partial(_router_body, n_steps=n_steps,
                          num_tokens=num_tokens),
        grid=grid,
        in_specs=[
            pl.BlockSpec((_TILE, hidden), lambda i: (i, 0)),
            pl.BlockSpec((hidden, _NUM_EXPERTS), lambda i: (0, 0)),
        ],
        out_specs=[
            pl.BlockSpec((_TILE, _TOP_K), lambda i: (i, 0)),
            pl.BlockSpec((_TILE, _TOP_K), lambda i: (i, 0)),
            pl.BlockSpec((1, 1), lambda i: (0, 0)),
        ],
        out_shape=[
            jax.ShapeDtypeStruct((num_tokens, _TOP_K), jnp.float32),
            jax.ShapeDtypeStruct((num_tokens, _TOP_K), jnp.int32),
            jax.ShapeDtypeStruct((1, 1), jnp.float32),
        ],
        scratch_shapes=[
            pltpu.VMEM((1, _NUM_EXPERTS), jnp.float32),
            pltpu.VMEM((1, _NUM_EXPERTS), jnp.float32),
        ],
    )(x, wt)

    return (weights.reshape(batch, seq, _TOP_K),
            experts.reshape(batch, seq, _TOP_K),
            loss[0, 0])


# speedup vs baseline: 2.4968x; 1.0684x over previous
import functools

import jax
import jax.numpy as jnp
from jax import lax
from jax.experimental import pallas as pl
from jax.experimental.pallas import tpu as pltpu

_NUM_EXPERTS = 64
_TOP_K = 2
_ALPHA = 0.01
_TILE = 4096


def _router_body(x_ref, wt_ref, w_out, e_out, loss_out, psum, cnt, *, n_steps,
                 num_tokens):
    pid = pl.program_id(0)

    @pl.when(pid == 0)
    def _init():
        psum[...] = jnp.zeros_like(psum)
        cnt[...] = jnp.zeros_like(cnt)

    logits = jnp.dot(x_ref[...], wt_ref[...],
                     preferred_element_type=jnp.float32)
    col = lax.broadcasted_iota(jnp.int32, logits.shape, 1)

    m1 = jnp.max(logits, axis=-1, keepdims=True)
    i1 = jnp.min(jnp.where(logits == m1, col, _NUM_EXPERTS), axis=-1,
                 keepdims=True)
    masked = jnp.where(col == i1, -jnp.inf, logits)
    m2 = jnp.max(masked, axis=-1, keepdims=True)
    i2 = jnp.min(jnp.where(masked == m2, col, _NUM_EXPERTS), axis=-1,
                 keepdims=True)

    t = jnp.exp(m2 - m1)
    w1 = 1.0 / (1.0 + t)
    w_out[...] = jnp.concatenate([w1, 1.0 - w1], axis=1)
    e_out[...] = jnp.concatenate([i1, i2], axis=1)

    e = jnp.exp(logits - m1)
    z = jnp.sum(e, axis=-1, keepdims=True)
    psum[...] += jnp.sum(e / z, axis=0, keepdims=True)
    hits = (col == i1).astype(jnp.float32) + (col == i2).astype(jnp.float32)
    cnt[...] += jnp.sum(hits, axis=0, keepdims=True)

    @pl.when(pid == n_steps - 1)
    def _fin():
        scale = _ALPHA * _NUM_EXPERTS / (num_tokens * num_tokens)
        loss_out[...] = scale * jnp.sum(psum[...] * cnt[...],
                                        keepdims=True)


def kernel(hidden_states, W_gate):
    batch, seq, hidden = hidden_states.shape
    num_tokens = batch * seq
    x = hidden_states.reshape(num_tokens, hidden)
    wt = W_gate.T
    n_steps = num_tokens // _TILE

    grid = (n_steps,)
    weights, experts, loss = pl.pallas_call(
        functools.partial(_router_body, n_steps=n_steps,
                          num_tokens=num_tokens),
        grid=grid,
        in_specs=[
            pl.BlockSpec((_TILE, hidden), lambda i: (i, 0)),
            pl.BlockSpec((hidden, _NUM_EXPERTS), lambda i: (0, 0)),
        ],
        out_specs=[
            pl.BlockSpec((_TILE, _TOP_K), lambda i: (i, 0)),
            pl.BlockSpec((_TILE, _TOP_K), lambda i: (i, 0)),
            pl.BlockSpec((1, 1), lambda i: (0, 0)),
        ],
        out_shape=[
            jax.ShapeDtypeStruct((num_tokens, _TOP_K), jnp.float32),
            jax.ShapeDtypeStruct((num_tokens, _TOP_K), jnp.int32),
            jax.ShapeDtypeStruct((1, 1), jnp.float32),
        ],
        scratch_shapes=[
            pltpu.VMEM((1, _NUM_EXPERTS), jnp.float32),
            pltpu.VMEM((1, _NUM_EXPERTS), jnp.float32),
        ],
    )(x, wt)

    return (weights.reshape(batch, seq, _TOP_K),
            experts.reshape(batch, seq, _TOP_K),
            loss[0, 0])
